# baseline (speedup 1.0000x reference)
import functools

import jax
import jax.numpy as jnp
from jax import lax
from jax.experimental import pallas as pl
from jax.experimental.pallas import tpu as pltpu
from jax.experimental.pallas import tpu_sc as plsc

D = 32
BATCH = 16384
BETA = 1.0
NC, NS, L = 2, 16, 16
NW = NC * NS
PAIRS_W = BATCH // NW
ROWS_W = 2 * PAIRS_W
IDX_CHUNK = 128
N_CHUNKS = ROWS_W // IDX_CHUNK
BLOCKS = PAIRS_W // L

_mesh = plsc.VectorSubcoreMesh(core_axis_name="c", subcore_axis_name="s")


@functools.partial(
    pl.kernel,
    out_type=jax.ShapeDtypeStruct((BATCH,), jnp.float32),
    mesh=_mesh,
    compiler_params=pltpu.CompilerParams(
        needs_layout_passes=False, use_tc_tiling_on_sc=False
    ),
    scratch_types=[
        pltpu.VMEM((N_CHUNKS, IDX_CHUNK), jnp.int32),
        pltpu.VMEM((ROWS_W, D), jnp.float32),
        pltpu.VMEM((PAIRS_W,), jnp.float32),
        pltpu.SemaphoreType.DMA,
    ],
)
def _lorentzian_sc(idx_hbm, table_hbm, out_hbm, idx_v, rows_v, out_v, sem):
    wid = lax.axis_index("s") * NC + lax.axis_index("c")

    pltpu.sync_copy(idx_hbm.at[pl.ds(wid * N_CHUNKS, N_CHUNKS)], idx_v)

    copies = []
    for j in range(N_CHUNKS):
        copies.append(
            pltpu.async_copy(
                table_hbm.at[idx_v.at[j]],
                rows_v.at[pl.ds(j * IDX_CHUNK, IDX_CHUNK)],
                sem,
            )
        )
    for c in copies:
        c.wait()

    def block(b, carry):
        p = b * L + lax.iota(jnp.int32, L)
        r_u = 2 * p
        r_v = r_u + 1
        acc_uv = jnp.zeros((L,), jnp.float32)
        acc_uu = jnp.zeros((L,), jnp.float32)
        acc_vv = jnp.zeros((L,), jnp.float32)
        for d in range(D):
            col = jnp.full((L,), d, jnp.int32)
            u = plsc.load_gather(rows_v, [r_u, col])
            v = plsc.load_gather(rows_v, [r_v, col])
            acc_uv = acc_uv + u * v
            acc_uu = acc_uu + u * u
            acc_vv = acc_vv + v * v
        prod = (BETA + acc_uu) * (BETA + acc_vv)
        i = plsc.bitcast(prod, jnp.int32)
        i = (i >> 1) + 0x1FBD1DF6
        y = plsc.bitcast(i, jnp.float32)
        y = 0.5 * (y + prod / y)
        y = 0.5 * (y + prod / y)
        y = 0.5 * (y + prod / y)
        out_v[pl.ds(b * L, L)] = 2.0 * y - 2.0 * acc_uv + (1e-5 - 2.0 * BETA)
        return carry

    lax.fori_loop(0, BLOCKS, block, 0)
    pltpu.sync_copy(out_v, out_hbm.at[pl.ds(wid * PAIRS_W, PAIRS_W)])


def kernel(idxs, table):
    idx2d = idxs.reshape(NW * N_CHUNKS, IDX_CHUNK)
    return _lorentzian_sc(idx2d, table)


# speedup vs baseline: 1.5363x; 1.5363x over previous
import functools

import jax
import jax.numpy as jnp
from jax import lax
from jax.experimental import pallas as pl
from jax.experimental.pallas import tpu as pltpu
from jax.experimental.pallas import tpu_sc as plsc

D = 32
BATCH = 16384
BETA = 1.0
NC, NS, L = 2, 16, 16
NW = NC * NS
PAIRS_W = BATCH // NW
ROWS_W = 2 * PAIRS_W
CHUNK = 128
N_CHUNKS = ROWS_W // CHUNK
BLOCKS = CHUNK // 2 // L

_mesh = plsc.VectorSubcoreMesh(core_axis_name="c", subcore_axis_name="s")


@functools.partial(
    pl.kernel,
    out_type=jax.ShapeDtypeStruct((BATCH,), jnp.float32),
    mesh=_mesh,
    compiler_params=pltpu.CompilerParams(needs_layout_passes=False),
    scratch_types=[
        pltpu.VMEM((N_CHUNKS, CHUNK), jnp.int32),
        pltpu.VMEM((CHUNK, D), jnp.float32),
        pltpu.VMEM((PAIRS_W,), jnp.float32),
        pltpu.SemaphoreType.DMA,
    ],
)
def _lorentzian_sc(idx_hbm, table_hbm, out_hbm, idx_v, rows_v, out_v, sem):
    wid = lax.axis_index("s") * NC + lax.axis_index("c")

    pltpu.sync_copy(idx_hbm.at[pl.ds(wid * N_CHUNKS, N_CHUNKS)], idx_v)

    def chunk_body(c, carry):
        copies = []
        for j in range(0, CHUNK, L):
            vec = idx_v[c, pl.ds(j, L)]
            for l in range(L):
                copies.append(
                    pltpu.async_copy(table_hbm.at[vec[l]], rows_v.at[j + l], sem)
                )
        for cp in copies:
            cp.wait()

        for b in range(BLOCKS):
            p = b * L + lax.iota(jnp.int32, L)
            acc_uv = jnp.zeros((L,), jnp.float32)
            acc_uu = jnp.zeros((L,), jnp.float32)
            acc_vv = jnp.zeros((L,), jnp.float32)
            for d in range(D):
                col = jnp.full((L,), d, jnp.int32)
                u = plsc.load_gather(rows_v, [2 * p, col])
                v = plsc.load_gather(rows_v, [2 * p + 1, col])
                acc_uv = acc_uv + u * v
                acc_uu = acc_uu + u * u
                acc_vv = acc_vv + v * v
            prod = (BETA + acc_uu) * (BETA + acc_vv)
            i32 = plsc.bitcast(prod, jnp.int32)
            i32 = (i32 >> 1) + 0x1FBD1DF6
            y = plsc.bitcast(i32, jnp.float32)
            y = 0.5 * (y + prod / y)
            y = 0.5 * (y + prod / y)
            y = 0.5 * (y + prod / y)
            out_v[pl.ds(c * (CHUNK // 2) + b * L, L)] = (
                2.0 * y - 2.0 * acc_uv + (1e-5 - 2.0 * BETA)
            )
        return carry

    lax.fori_loop(0, N_CHUNKS, chunk_body, 0)
    pltpu.sync_copy(out_v, out_hbm.at[pl.ds(wid * PAIRS_W, PAIRS_W)])


def kernel(idxs, table):
    idx2d = idxs.reshape(NW * N_CHUNKS, CHUNK)
    return _lorentzian_sc(idx2d, table)
